# baseline (speedup 1.0000x reference)
import functools

import jax
import jax.numpy as jnp
from jax import lax
from jax.experimental import pallas as pl
from jax.experimental.pallas import tpu as pltpu
from jax.experimental.pallas import tpu_sc as plsc

_N = 100000
_ZMAX = 87
_F = 128

_NC = 2
_NS = 16
_NW = _NC * _NS
_C = 128
_K = 25
_N_PAD = _NW * _K * _C


def _table_body(emb_ref, ec_ref, cw_ref, out_ref):
    out_ref[...] = emb_ref[...] + lax.dot_general(
        ec_ref[...], cw_ref[...], (((1,), (1,)), ((), ())),
        preferred_element_type=jnp.float32)


_table_call = pl.pallas_call(
    _table_body,
    out_shape=jax.ShapeDtypeStruct((_ZMAX, _F), jnp.float32),
)


_sc_mesh = plsc.VectorSubcoreMesh(core_axis_name="c", subcore_axis_name="s")


@functools.partial(
    pl.kernel,
    mesh=_sc_mesh,
    out_type=jax.ShapeDtypeStruct((_N_PAD, _F), jnp.float32),
    scratch_types=[
        pltpu.VMEM((_K, _C), jnp.int32),
        pltpu.VMEM((_C, _F), jnp.float32),
        pltpu.SemaphoreType.DMA,
    ],
)
def _gather_kernel(table_hbm, z_hbm, out_hbm, idx_v, rows_v, sem):
    wid = lax.axis_index("s") * _NC + lax.axis_index("c")
    pltpu.sync_copy(z_hbm.at[wid], idx_v)
    base = wid * (_K * _C)

    def body(c, carry):
        pltpu.async_copy(table_hbm.at[idx_v.at[c]], rows_v, sem).wait()
        pltpu.sync_copy(rows_v, out_hbm.at[pl.ds(base + c * _C, _C)])
        return carry

    lax.fori_loop(0, _K, body, 0)


def kernel(Z, element_embedding, electron_config, config_weight):
    table = _table_call(element_embedding, electron_config, config_weight)
    z_pad = jnp.pad(Z.astype(jnp.int32), (0, _N_PAD - _N)).reshape(_NW, _K, _C)
    out = _gather_kernel(table, z_pad)
    return out[:_N]


# speedup vs baseline: 1.0623x; 1.0623x over previous
import functools

import jax
import jax.numpy as jnp
from jax import lax
from jax.experimental import pallas as pl
from jax.experimental.pallas import tpu as pltpu
from jax.experimental.pallas import tpu_sc as plsc

_N = 100000
_ZMAX = 87
_F = 128

_NC = 2
_NS = 16
_NW = _NC * _NS
_C = 128
_K = 25
_N_PAD = _NW * _K * _C


def _table_body(emb_ref, ec_ref, cw_ref, out_ref):
    out_ref[...] = emb_ref[...] + lax.dot_general(
        ec_ref[...], cw_ref[...], (((1,), (1,)), ((), ())),
        preferred_element_type=jnp.float32)


_table_call = pl.pallas_call(
    _table_body,
    out_shape=jax.ShapeDtypeStruct((_ZMAX, _F), jnp.float32),
)


_sc_mesh = plsc.VectorSubcoreMesh(core_axis_name="c", subcore_axis_name="s")


_NBUF = 4


@functools.partial(
    pl.kernel,
    mesh=_sc_mesh,
    out_type=jax.ShapeDtypeStruct((_N_PAD, _F), jnp.float32),
    scratch_types=[
        pltpu.VMEM((_K, _C), jnp.int32),
    ]
    + [pltpu.VMEM((_C, _F), jnp.float32) for _ in range(_NBUF)]
    + [pltpu.SemaphoreType.DMA for _ in range(2 * _NBUF)],
)
def _gather_kernel(table_hbm, z_hbm, out_hbm, idx_v, *bufs_and_sems):
    rows = bufs_and_sems[:_NBUF]
    gsem = bufs_and_sems[_NBUF:2 * _NBUF]
    wsem = bufs_and_sems[2 * _NBUF:]
    wid = lax.axis_index("s") * _NC + lax.axis_index("c")
    pltpu.sync_copy(z_hbm.at[wid], idx_v)
    base = wid * (_K * _C)

    def fire_gather(c, b):
        pltpu.async_copy(table_hbm.at[idx_v.at[c]], rows[b], gsem[b])

    def wait_gather(c, b):
        pltpu.make_async_copy(table_hbm.at[idx_v.at[c]], rows[b], gsem[b]).wait()

    def out_slice(c):
        return out_hbm.at[pl.ds(base + c * _C, _C)]

    for b in range(_NBUF):
        fire_gather(b, b)

    def body(j, carry):
        for b in range(_NBUF):
            c = j * _NBUF + b
            wait_gather(c, b)
            pltpu.async_copy(rows[b], out_slice(c), wsem[b])
        for b in range(_NBUF):
            c = j * _NBUF + b
            pltpu.make_async_copy(rows[b], out_slice(c), wsem[b]).wait()

            @pl.when(c + _NBUF < _K)
            def _():
                fire_gather(c + _NBUF, b)

        return carry

    lax.fori_loop(0, _K // _NBUF, body, 0)

    for b in range(_K % _NBUF):
        c = (_K // _NBUF) * _NBUF + b
        wait_gather(c, b)
        pltpu.async_copy(rows[b], out_slice(c), wsem[b])
    for b in range(_K % _NBUF):
        c = (_K // _NBUF) * _NBUF + b
        pltpu.make_async_copy(rows[b], out_slice(c), wsem[b]).wait()


def kernel(Z, element_embedding, electron_config, config_weight):
    table = _table_call(element_embedding, electron_config, config_weight)
    z_pad = jnp.pad(Z.astype(jnp.int32), (0, _N_PAD - _N)).reshape(_NW, _K, _C)
    out = _gather_kernel(table, z_pad)
    return out[:_N]


# speedup vs baseline: 1.6587x; 1.5615x over previous
import functools

import jax
import jax.numpy as jnp
from jax import lax
from jax.experimental import pallas as pl
from jax.experimental.pallas import tpu as pltpu
from jax.experimental.pallas import tpu_sc as plsc

_N = 100000
_ZMAX = 87
_F = 128

_NC = 2
_NS = 16
_NW = _NC * _NS
_C = 128
_K = 25
_W = _K * _C
_LAST_BASE = _N - _W


def _table_body(emb_ref, ec_ref, cw_ref, out_ref):
    out_ref[...] = emb_ref[...] + lax.dot_general(
        ec_ref[...], cw_ref[...], (((1,), (1,)), ((), ())),
        preferred_element_type=jnp.float32)


_table_call = pl.pallas_call(
    _table_body,
    out_shape=jax.ShapeDtypeStruct((_ZMAX, _F), jnp.float32),
)


_sc_mesh = plsc.VectorSubcoreMesh(core_axis_name="c", subcore_axis_name="s")


_NBUF = 4


@functools.partial(
    pl.kernel,
    mesh=_sc_mesh,
    out_type=jax.ShapeDtypeStruct((_N, _F), jnp.float32),
    scratch_types=[
        pltpu.VMEM((_K, _C), jnp.int32),
    ]
    + [pltpu.VMEM((_C, _F), jnp.float32) for _ in range(_NBUF)]
    + [pltpu.SemaphoreType.DMA for _ in range(2 * _NBUF)],
)
def _gather_kernel(table_hbm, z_hbm, out_hbm, idx_v, *bufs_and_sems):
    rows = bufs_and_sems[:_NBUF]
    gsem = bufs_and_sems[_NBUF:2 * _NBUF]
    wsem = bufs_and_sems[2 * _NBUF:]
    wid = lax.axis_index("s") * _NC + lax.axis_index("c")
    pltpu.sync_copy(z_hbm.at[wid], idx_v)
    base = lax.min(wid * _W, _LAST_BASE)

    def fire_gather(c, b):
        pltpu.async_copy(table_hbm.at[idx_v.at[c]], rows[b], gsem[b])

    def wait_gather(c, b):
        pltpu.make_async_copy(table_hbm.at[idx_v.at[c]], rows[b], gsem[b]).wait()

    def out_slice(c):
        return out_hbm.at[pl.ds(base + c * _C, _C)]

    for b in range(_NBUF):
        fire_gather(b, b)

    def body(j, carry):
        for b in range(_NBUF):
            c = j * _NBUF + b
            wait_gather(c, b)
            pltpu.async_copy(rows[b], out_slice(c), wsem[b])
        for b in range(_NBUF):
            c = j * _NBUF + b
            pltpu.make_async_copy(rows[b], out_slice(c), wsem[b]).wait()

            @pl.when(c + _NBUF < _K)
            def _():
                fire_gather(c + _NBUF, b)

        return carry

    lax.fori_loop(0, _K // _NBUF, body, 0)

    for b in range(_K % _NBUF):
        c = (_K // _NBUF) * _NBUF + b
        wait_gather(c, b)
        pltpu.async_copy(rows[b], out_slice(c), wsem[b])
    for b in range(_K % _NBUF):
        c = (_K // _NBUF) * _NBUF + b
        pltpu.make_async_copy(rows[b], out_slice(c), wsem[b]).wait()


def kernel(Z, element_embedding, electron_config, config_weight):
    table = _table_call(element_embedding, electron_config, config_weight)
    z32 = Z.astype(jnp.int32)
    z_slabs = jnp.concatenate(
        [z32[: (_NW - 1) * _W], z32[_LAST_BASE:]]
    ).reshape(_NW, _K, _C)
    return _gather_kernel(table, z_slabs)
